# baseline (speedup 1.0000x reference)
import jax
import jax.numpy as jnp
from jax.experimental import pallas as pl

BM = 400


def _pass1(a_ref, x_ref, w1_ref, b1_ref, w2_ref, s2_ref):
    a = a_ref[...].astype(jnp.bfloat16)
    t = jnp.dot(a, x_ref[...], preferred_element_type=jnp.float32)
    h = jnp.dot(t, w1_ref[...], preferred_element_type=jnp.float32,
                precision=jax.lax.Precision.HIGHEST)
    h = jnp.maximum(h + b1_ref[...], 0.0)
    s2 = jnp.dot(h, w2_ref[...], preferred_element_type=jnp.float32,
                 precision=jax.lax.Precision.HIGHEST)
    s2_ref[...] = s2.astype(jnp.bfloat16)


def _pass2(a_ref, s2_ref, b2_ref, out_ref):
    a = a_ref[...].astype(jnp.bfloat16)
    t = jnp.dot(a, s2_ref[...], preferred_element_type=jnp.float32)
    out_ref[...] = t + b2_ref[...]


def kernel(x, adj, W1, b1, W2, b2):
    n, nfeat = x.shape
    h1 = W1.shape[1]
    nhid = W2.shape[1]
    grid = (n // BM,)

    x_bf = x.astype(jnp.bfloat16)
    b1_2d = b1.reshape(1, h1)
    b2_2d = b2.reshape(1, nhid)

    s2 = pl.pallas_call(
        _pass1,
        grid=grid,
        in_specs=[
            pl.BlockSpec((BM, n), lambda i: (i, 0)),
            pl.BlockSpec((n, nfeat), lambda i: (0, 0)),
            pl.BlockSpec((nfeat, h1), lambda i: (0, 0)),
            pl.BlockSpec((1, h1), lambda i: (0, 0)),
            pl.BlockSpec((h1, nhid), lambda i: (0, 0)),
        ],
        out_specs=pl.BlockSpec((BM, nhid), lambda i: (i, 0)),
        out_shape=jax.ShapeDtypeStruct((n, nhid), jnp.bfloat16),
    )(adj, x_bf, W1, b1_2d, W2)

    out = pl.pallas_call(
        _pass2,
        grid=grid,
        in_specs=[
            pl.BlockSpec((BM, n), lambda i: (i, 0)),
            pl.BlockSpec((n, nhid), lambda i: (0, 0)),
            pl.BlockSpec((1, nhid), lambda i: (0, 0)),
        ],
        out_specs=pl.BlockSpec((BM, nhid), lambda i: (i, 0)),
        out_shape=jax.ShapeDtypeStruct((n, nhid), jnp.float32),
    )(adj, s2, b2_2d)

    return out
